# baseline (speedup 1.0000x reference)
import functools

import jax
import jax.numpy as jnp
import numpy as np
from jax import lax
from jax.experimental import pallas as pl
from jax.experimental.pallas import tpu as pltpu
from jax.experimental.pallas import tpu_sc as plsc

M = 200000
B = 16384
D = 128
NW = 32
BLK = 6248
TAIL_START = NW * BLK
TAIL = M - TAIL_START
JB = B // NW
CH = 128
ECH = JB // CH
CR = 200
OFFS = [i * CR for i in range(BLK // CR)] + [BLK - CR]
NCHUNK = len(OFFS)
DEPTH = 3
PD = 2
ICH0 = 3128
ICH1 = 3120


def _rotl(v, d):
    return ((v << np.uint32(d)) | (v >> np.uint32(32 - d))).astype(np.uint32)


def _threefry2x32(k0, k1, x0, x1):
    rotations = ((13, 15, 26, 6), (17, 29, 16, 24))
    k0 = np.uint32(k0)
    k1 = np.uint32(k1)
    ks = (k0, k1, np.uint32(k0 ^ k1 ^ np.uint32(0x1BD11BDA)))
    x0 = (x0 + ks[0]).astype(np.uint32)
    x1 = (x1 + ks[1]).astype(np.uint32)
    for r in range(5):
        for rot in rotations[r % 2]:
            x0 = (x0 + x1).astype(np.uint32)
            x1 = _rotl(x1, rot)
            x1 = x0 ^ x1
        x0 = (x0 + ks[(r + 1) % 3]).astype(np.uint32)
        x1 = (x1 + ks[(r + 2) % 3] + np.uint32(r + 1)).astype(np.uint32)
    return x0, x1


def _np_split(kd):
    b1, b2 = _threefry2x32(
        kd[0], kd[1], np.zeros(2, np.uint32), np.arange(2, dtype=np.uint32))
    return np.stack([b1, b2], axis=1)


def _np_bits32(kd, n):
    b1, b2 = _threefry2x32(
        kd[0], kd[1], np.zeros(n, np.uint32), np.arange(n, dtype=np.uint32))
    return b1 ^ b2


def _np_permutation(seed, n):
    kd = np.array([seed >> 32, seed & 0xFFFFFFFF], np.uint32)
    x = np.arange(n, dtype=np.int32)
    num_rounds = int(np.ceil(3 * np.log(max(1, n)) / np.log(2**32 - 1)))
    for _ in range(num_rounds):
        ks = _np_split(kd)
        kd, sub = ks[0], ks[1]
        x = x[np.argsort(_np_bits32(sub, n), kind="stable")]
    return x


@functools.lru_cache(maxsize=None)
def _plan():
    swap = _np_permutation(42, M)[:B].astype(np.int32)
    owner = np.where(swap >= TAIL_START, 0, swap // BLK)
    order = np.argsort(owner, kind="stable").astype(np.int32)
    dst_sorted = swap[order]
    counts = np.bincount(owner, minlength=NW)
    assert counts.min() > 0
    kmax = int(counts.max())
    nch = -(-kmax // CH)
    k = nch * CH
    scat_src = np.zeros((NW, nch, CH), np.int32)
    scat_dst = np.zeros((NW, nch, CH), np.int32)
    offs = np.concatenate([[0], np.cumsum(counts)])
    for w in range(NW):
        s, e = int(offs[w]), int(offs[w + 1])
        seg_src = order[s:e]
        seg_dst = dst_sorted[s:e]
        pad = np.arange(k - (e - s)) % (e - s)
        seg_src = np.concatenate([seg_src, seg_src[pad]]).astype(np.int32)
        seg_dst = np.concatenate([seg_dst, seg_dst[pad]]).astype(np.int32)
        scat_src[w] = seg_src.reshape(nch, CH)
        scat_dst[w] = seg_dst.reshape(nch, CH)
    evict = swap.reshape(NW, ECH, CH)
    return scat_src, scat_dst, evict, nch


def _make_kernel(nch, int_dtype):
    mesh = plsc.VectorSubcoreMesh(core_axis_name="c", subcore_axis_name="s")
    info = plsc.get_sparse_core_info()
    ncores = info.num_cores
    K = nch * CH

    def body(ssrc_h, sdst_h, ev_h,
             bx, by, bt, bidx, in_x, in_y, in_t, in_idx,
             ox, oy, ot, oidx,
             ssrc_v, sdst_v, ev_v, cb, pbuf, ib0, ib1, tbuf,
             gy, gt, gi, ey, et, ei,
             r0, r1, r2, r3, w0, w1, w2, w3, g0, g1, g2, v0, v1, v2,
             s_g, s_ie, s_ir, s_iw0, s_iw1, s_s):
        wid = lax.axis_index("s") * ncores + lax.axis_index("c")
        base = wid * BLK
        rsem = (r0, r1, r2, r3)
        wsem = (w0, w1, w2, w3)
        gsem = (g0, g1, g2)
        vsem = (v0, v1, v2)

        pltpu.sync_copy(ssrc_h.at[wid], ssrc_v)
        pltpu.sync_copy(sdst_h.at[wid], sdst_v)
        pltpu.sync_copy(ev_h.at[wid], ev_v)

        gint = []
        for ch in range(nch):
            sl = pl.ds(ch * CH, CH)
            gint.append(pltpu.async_copy(in_y.at[ssrc_v.at[ch]], gy.at[sl], s_g))
            gint.append(pltpu.async_copy(in_t.at[ssrc_v.at[ch]], gt.at[sl], s_g))
            gint.append(pltpu.async_copy(in_idx.at[ssrc_v.at[ch]], gi.at[sl], s_g))
        for ch in range(ECH):
            sl = pl.ds(ch * CH, CH)
            gint.append(pltpu.async_copy(by.at[ev_v.at[ch]], ey.at[sl], s_ie))
            gint.append(pltpu.async_copy(bt.at[ev_v.at[ch]], et.at[sl], s_ie))
            gint.append(pltpu.async_copy(bidx.at[ev_v.at[ch]], ei.at[sl], s_ie))

        rd = [None] * DEPTH
        wr = [None] * DEPTH
        for i in range(NCHUNK):
            s = i % DEPTH
            if wr[s] is not None:
                wr[s].wait()
                wr[s] = None
            rd[s] = pltpu.async_copy(
                bx.at[pl.ds(base + OFFS[i], CR)], cb.at[s], rsem[s])
            j = i - (DEPTH - 1)
            if j >= 0:
                sj = j % DEPTH
                rd[sj].wait()
                rd[sj] = None
                wr[sj] = pltpu.async_copy(
                    cb.at[sj], ox.at[pl.ds(base + OFFS[j], CR)], wsem[sj])
        for j in range(max(0, NCHUNK - DEPTH + 1), NCHUNK):
            sj = j % DEPTH
            rd[sj].wait()
            rd[sj] = None
            wr[sj] = pltpu.async_copy(
                cb.at[sj], ox.at[pl.ds(base + OFFS[j], CR)], wsem[sj])
        for s in range(DEPTH):
            if wr[s] is not None:
                wr[s].wait()

        @pl.when(wid == 0)
        def _tail():
            sl = pl.ds(TAIL_START, TAIL)
            pltpu.sync_copy(bx.at[sl], ox.at[sl])
            for src, dst in ((by, oy), (bt, ot), (bidx, oidx)):
                pltpu.sync_copy(src.at[sl], tbuf)
                pltpu.sync_copy(tbuf, dst.at[sl])

        jobs = [("ev", ch) for ch in range(ECH)] + \
               [("sc", ch) for ch in range(nch)]

        def g_issue(kind, ch, s):
            if kind == "ev":
                return pltpu.async_copy(bx.at[ev_v.at[ch]], pbuf.at[s], gsem[s])
            return pltpu.async_copy(in_x.at[ssrc_v.at[ch]], pbuf.at[s], gsem[s])

        def w_issue(kind, ch, s):
            if kind == "ev":
                return pltpu.async_copy(
                    pbuf.at[s], ox.at[pl.ds(M + wid * JB + ch * CH, CH)],
                    vsem[s])
            return pltpu.async_copy(pbuf.at[s], ox.at[sdst_v.at[ch]], vsem[s])

        gd = [None] * PD
        wd = [None] * PD
        for t, (kind, ch) in enumerate(jobs):
            s = t % PD
            if wd[s] is not None:
                wd[s].wait()
                wd[s] = None
            gd[s] = g_issue(kind, ch, s)
            u = t - (PD - 1)
            if u >= 0:
                su = u % PD
                gd[su].wait()
                gd[su] = None
                wd[su] = w_issue(jobs[u][0], jobs[u][1], su)
        for u in range(max(0, len(jobs) - PD + 1), len(jobs)):
            su = u % PD
            gd[su].wait()
            gd[su] = None
            wd[su] = w_issue(jobs[u][0], jobs[u][1], su)
        for s in range(PD):
            if wd[s] is not None:
                wd[s].wait()

        for d in gint:
            d.wait()
        esl = pl.ds(M + wid * JB, JB)
        pltpu.sync_copy(ey, oy.at[esl])
        pltpu.sync_copy(et, ot.at[esl])
        pltpu.sync_copy(ei, oidx.at[esl])

        ibs = (ib0, ib1)
        isz = (ICH0, ICH1)
        ioff = (0, ICH0)
        iwsems = (s_iw0, s_iw1)
        iw = [None, None]
        for src, dst in ((by, oy), (bt, ot), (bidx, oidx)):
            for c in range(2):
                if iw[c] is not None:
                    iw[c].wait()
                pltpu.async_copy(
                    src.at[pl.ds(base + ioff[c], isz[c])], ibs[c], s_ir).wait()
                iw[c] = pltpu.async_copy(
                    ibs[c], dst.at[pl.ds(base + ioff[c], isz[c])], iwsems[c])
        for d in iw:
            if d is not None:
                d.wait()

        scat = []
        for ch in range(nch):
            sl = pl.ds(ch * CH, CH)
            scat.append(pltpu.async_copy(gy.at[sl], oy.at[sdst_v.at[ch]], s_s))
            scat.append(pltpu.async_copy(gt.at[sl], ot.at[sdst_v.at[ch]], s_s))
            scat.append(pltpu.async_copy(gi.at[sl], oidx.at[sdst_v.at[ch]], s_s))
        for d in scat:
            d.wait()

    out_type = (
        jax.ShapeDtypeStruct((M + B, D), jnp.float32),
        jax.ShapeDtypeStruct((M + B,), int_dtype),
        jax.ShapeDtypeStruct((M + B,), int_dtype),
        jax.ShapeDtypeStruct((M + B,), int_dtype),
    )
    scratch = [
        pltpu.VMEM((nch, CH), jnp.int32),
        pltpu.VMEM((nch, CH), jnp.int32),
        pltpu.VMEM((ECH, CH), jnp.int32),
        pltpu.VMEM((DEPTH, CR, D), jnp.float32),
        pltpu.VMEM((PD, CH, D), jnp.float32),
        pltpu.VMEM((ICH0,), int_dtype),
        pltpu.VMEM((ICH1,), int_dtype),
        pltpu.VMEM((TAIL,), int_dtype),
        pltpu.VMEM((K,), int_dtype),
        pltpu.VMEM((K,), int_dtype),
        pltpu.VMEM((K,), int_dtype),
        pltpu.VMEM((JB,), int_dtype),
        pltpu.VMEM((JB,), int_dtype),
        pltpu.VMEM((JB,), int_dtype),
    ] + [pltpu.SemaphoreType.DMA] * 20
    return pl.kernel(body, out_type=out_type, mesh=mesh, scratch_types=scratch)


_SCAT_SRC, _SCAT_DST, _EVICT, _NCH = _plan()


def kernel(bx, by, bt, bidx, in_x, in_y, in_t, in_idx):
    k = _make_kernel(_NCH, by.dtype)
    return k(jnp.asarray(_SCAT_SRC), jnp.asarray(_SCAT_DST),
             jnp.asarray(_EVICT),
             bx, by, bt, bidx, in_x, in_y, in_t, in_idx)


# speedup vs baseline: 1.0208x; 1.0208x over previous
import functools

import jax
import jax.numpy as jnp
import numpy as np
from jax import lax
from jax.experimental import pallas as pl
from jax.experimental.pallas import tpu as pltpu
from jax.experimental.pallas import tpu_sc as plsc

M = 200000
B = 16384
D = 128
NW = 32
BLK = 6248
TAIL_START = NW * BLK
TAIL = M - TAIL_START
JB = B // NW
CH = 128
ECH = JB // CH
CR = 88
NCHUNK = BLK // CR
DEPTH = 4
PD = 3
ICH0 = 3128
ICH1 = 3120


def _rotl(v, d):
    return ((v << np.uint32(d)) | (v >> np.uint32(32 - d))).astype(np.uint32)


def _threefry2x32(k0, k1, x0, x1):
    rotations = ((13, 15, 26, 6), (17, 29, 16, 24))
    k0 = np.uint32(k0)
    k1 = np.uint32(k1)
    ks = (k0, k1, np.uint32(k0 ^ k1 ^ np.uint32(0x1BD11BDA)))
    x0 = (x0 + ks[0]).astype(np.uint32)
    x1 = (x1 + ks[1]).astype(np.uint32)
    for r in range(5):
        for rot in rotations[r % 2]:
            x0 = (x0 + x1).astype(np.uint32)
            x1 = _rotl(x1, rot)
            x1 = x0 ^ x1
        x0 = (x0 + ks[(r + 1) % 3]).astype(np.uint32)
        x1 = (x1 + ks[(r + 2) % 3] + np.uint32(r + 1)).astype(np.uint32)
    return x0, x1


def _np_split(kd):
    b1, b2 = _threefry2x32(
        kd[0], kd[1], np.zeros(2, np.uint32), np.arange(2, dtype=np.uint32))
    return np.stack([b1, b2], axis=1)


def _np_bits32(kd, n):
    b1, b2 = _threefry2x32(
        kd[0], kd[1], np.zeros(n, np.uint32), np.arange(n, dtype=np.uint32))
    return b1 ^ b2


def _np_permutation(seed, n):
    kd = np.array([seed >> 32, seed & 0xFFFFFFFF], np.uint32)
    x = np.arange(n, dtype=np.int32)
    num_rounds = int(np.ceil(3 * np.log(max(1, n)) / np.log(2**32 - 1)))
    for _ in range(num_rounds):
        ks = _np_split(kd)
        kd, sub = ks[0], ks[1]
        x = x[np.argsort(_np_bits32(sub, n), kind="stable")]
    return x


@functools.lru_cache(maxsize=None)
def _plan():
    swap = _np_permutation(42, M)[:B].astype(np.int32)
    owner = np.where(swap >= TAIL_START, 0, swap // BLK)
    order = np.argsort(owner, kind="stable").astype(np.int32)
    dst_sorted = swap[order]
    counts = np.bincount(owner, minlength=NW)
    assert counts.min() > 0
    kmax = int(counts.max())
    nch = -(-kmax // CH)
    k = nch * CH
    scat_src = np.zeros((NW, nch, CH), np.int32)
    scat_dst = np.zeros((NW, nch, CH), np.int32)
    offs = np.concatenate([[0], np.cumsum(counts)])
    for w in range(NW):
        s, e = int(offs[w]), int(offs[w + 1])
        seg_src = order[s:e]
        seg_dst = dst_sorted[s:e]
        pad = np.arange(k - (e - s)) % (e - s)
        seg_src = np.concatenate([seg_src, seg_src[pad]]).astype(np.int32)
        seg_dst = np.concatenate([seg_dst, seg_dst[pad]]).astype(np.int32)
        scat_src[w] = seg_src.reshape(nch, CH)
        scat_dst[w] = seg_dst.reshape(nch, CH)
    evict = swap.reshape(NW, ECH, CH)
    return scat_src, scat_dst, evict, nch


def _make_kernel(nch, int_dtype):
    mesh = plsc.VectorSubcoreMesh(core_axis_name="c", subcore_axis_name="s")
    info = plsc.get_sparse_core_info()
    ncores = info.num_cores
    K = nch * CH

    def body(ssrc_h, sdst_h, ev_h,
             bx, by, bt, bidx, in_x, in_y, in_t, in_idx,
             ox, oy, ot, oidx,
             ssrc_v, sdst_v, ev_v, cb, pbuf, ib0, ib1, tbuf,
             gy, gt, gi, ey, et, ei,
             r0, r1, r2, r3, w0, w1, w2, w3, g0, g1, g2, v0, v1, v2,
             s_g, s_ie, s_ir, s_iw0, s_iw1, s_s):
        wid = lax.axis_index("s") * ncores + lax.axis_index("c")
        base = wid * BLK
        rsem = (r0, r1, r2, r3)
        wsem = (w0, w1, w2, w3)
        gsem = (g0, g1, g2)
        vsem = (v0, v1, v2)

        pltpu.sync_copy(ssrc_h.at[wid], ssrc_v)
        pltpu.sync_copy(sdst_h.at[wid], sdst_v)
        pltpu.sync_copy(ev_h.at[wid], ev_v)

        gint = []
        for ch in range(nch):
            sl = pl.ds(ch * CH, CH)
            gint.append(pltpu.async_copy(in_y.at[ssrc_v.at[ch]], gy.at[sl], s_g))
            gint.append(pltpu.async_copy(in_t.at[ssrc_v.at[ch]], gt.at[sl], s_g))
            gint.append(pltpu.async_copy(in_idx.at[ssrc_v.at[ch]], gi.at[sl], s_g))
        for ch in range(ECH):
            sl = pl.ds(ch * CH, CH)
            gint.append(pltpu.async_copy(by.at[ev_v.at[ch]], ey.at[sl], s_ie))
            gint.append(pltpu.async_copy(bt.at[ev_v.at[ch]], et.at[sl], s_ie))
            gint.append(pltpu.async_copy(bidx.at[ev_v.at[ch]], ei.at[sl], s_ie))

        rd = [None] * DEPTH
        wr = [None] * DEPTH
        for i in range(NCHUNK):
            s = i % DEPTH
            if wr[s] is not None:
                wr[s].wait()
                wr[s] = None
            rd[s] = pltpu.async_copy(
                bx.at[pl.ds(base + i * CR, CR)], cb.at[s], rsem[s])
            j = i - (DEPTH - 1)
            if j >= 0:
                sj = j % DEPTH
                rd[sj].wait()
                rd[sj] = None
                wr[sj] = pltpu.async_copy(
                    cb.at[sj], ox.at[pl.ds(base + j * CR, CR)], wsem[sj])
        for j in range(max(0, NCHUNK - DEPTH + 1), NCHUNK):
            sj = j % DEPTH
            rd[sj].wait()
            rd[sj] = None
            wr[sj] = pltpu.async_copy(
                cb.at[sj], ox.at[pl.ds(base + j * CR, CR)], wsem[sj])
        for s in range(DEPTH):
            if wr[s] is not None:
                wr[s].wait()

        @pl.when(wid == 0)
        def _tail():
            sl = pl.ds(TAIL_START, TAIL)
            pltpu.sync_copy(bx.at[sl], ox.at[sl])
            for src, dst in ((by, oy), (bt, ot), (bidx, oidx)):
                pltpu.sync_copy(src.at[sl], tbuf)
                pltpu.sync_copy(tbuf, dst.at[sl])

        jobs = [("ev", ch) for ch in range(ECH)] + \
               [("sc", ch) for ch in range(nch)]

        def g_issue(kind, ch, s):
            if kind == "ev":
                return pltpu.async_copy(bx.at[ev_v.at[ch]], pbuf.at[s], gsem[s])
            return pltpu.async_copy(in_x.at[ssrc_v.at[ch]], pbuf.at[s], gsem[s])

        def w_issue(kind, ch, s):
            if kind == "ev":
                return pltpu.async_copy(
                    pbuf.at[s], ox.at[pl.ds(M + wid * JB + ch * CH, CH)],
                    vsem[s])
            return pltpu.async_copy(pbuf.at[s], ox.at[sdst_v.at[ch]], vsem[s])

        gd = [None] * PD
        wd = [None] * PD
        for t, (kind, ch) in enumerate(jobs):
            s = t % PD
            if wd[s] is not None:
                wd[s].wait()
                wd[s] = None
            gd[s] = g_issue(kind, ch, s)
            u = t - (PD - 1)
            if u >= 0:
                su = u % PD
                gd[su].wait()
                gd[su] = None
                wd[su] = w_issue(jobs[u][0], jobs[u][1], su)
        for u in range(max(0, len(jobs) - PD + 1), len(jobs)):
            su = u % PD
            gd[su].wait()
            gd[su] = None
            wd[su] = w_issue(jobs[u][0], jobs[u][1], su)
        for s in range(PD):
            if wd[s] is not None:
                wd[s].wait()

        for d in gint:
            d.wait()
        esl = pl.ds(M + wid * JB, JB)
        pltpu.sync_copy(ey, oy.at[esl])
        pltpu.sync_copy(et, ot.at[esl])
        pltpu.sync_copy(ei, oidx.at[esl])

        ibs = (ib0, ib1)
        isz = (ICH0, ICH1)
        ioff = (0, ICH0)
        iwsems = (s_iw0, s_iw1)
        iw = [None, None]
        for src, dst in ((by, oy), (bt, ot), (bidx, oidx)):
            for c in range(2):
                if iw[c] is not None:
                    iw[c].wait()
                pltpu.async_copy(
                    src.at[pl.ds(base + ioff[c], isz[c])], ibs[c], s_ir).wait()
                iw[c] = pltpu.async_copy(
                    ibs[c], dst.at[pl.ds(base + ioff[c], isz[c])], iwsems[c])
        for d in iw:
            if d is not None:
                d.wait()

        scat = []
        for ch in range(nch):
            sl = pl.ds(ch * CH, CH)
            scat.append(pltpu.async_copy(gy.at[sl], oy.at[sdst_v.at[ch]], s_s))
            scat.append(pltpu.async_copy(gt.at[sl], ot.at[sdst_v.at[ch]], s_s))
            scat.append(pltpu.async_copy(gi.at[sl], oidx.at[sdst_v.at[ch]], s_s))
        for d in scat:
            d.wait()

    out_type = (
        jax.ShapeDtypeStruct((M + B, D), jnp.float32),
        jax.ShapeDtypeStruct((M + B,), int_dtype),
        jax.ShapeDtypeStruct((M + B,), int_dtype),
        jax.ShapeDtypeStruct((M + B,), int_dtype),
    )
    scratch = [
        pltpu.VMEM((nch, CH), jnp.int32),
        pltpu.VMEM((nch, CH), jnp.int32),
        pltpu.VMEM((ECH, CH), jnp.int32),
        pltpu.VMEM((DEPTH, CR, D), jnp.float32),
        pltpu.VMEM((PD, CH, D), jnp.float32),
        pltpu.VMEM((ICH0,), int_dtype),
        pltpu.VMEM((ICH1,), int_dtype),
        pltpu.VMEM((TAIL,), int_dtype),
        pltpu.VMEM((K,), int_dtype),
        pltpu.VMEM((K,), int_dtype),
        pltpu.VMEM((K,), int_dtype),
        pltpu.VMEM((JB,), int_dtype),
        pltpu.VMEM((JB,), int_dtype),
        pltpu.VMEM((JB,), int_dtype),
    ] + [pltpu.SemaphoreType.DMA] * 20
    return pl.kernel(body, out_type=out_type, mesh=mesh, scratch_types=scratch)


_SCAT_SRC, _SCAT_DST, _EVICT, _NCH = _plan()


def kernel(bx, by, bt, bidx, in_x, in_y, in_t, in_idx):
    k = _make_kernel(_NCH, by.dtype)
    return k(jnp.asarray(_SCAT_SRC), jnp.asarray(_SCAT_DST),
             jnp.asarray(_EVICT),
             bx, by, bt, bidx, in_x, in_y, in_t, in_idx)
